# Initial kernel scaffold; baseline (speedup 1.0000x reference)
import jax
import jax.numpy as jnp
from jax.experimental import pallas as pl


def kernel(x, edge_attr, enn_w1, enn_b1, enn_w2, enn_b2, root_w, nn_bias, conv_ws, conv_bs, mlp_ws, mlp_bs, edge_index):
    raise NotImplementedError("write your pallas kernel here")


# speedup vs baseline: 3.3830x; 3.3830x over previous
import functools

import jax
import jax.numpy as jnp
from jax import lax
from jax.experimental import pallas as pl
from jax.experimental.pallas import tpu as pltpu
from jax.experimental.pallas import tpu_sc as plsc

N = 10000
E = 320000
IN_CH = 13
EMB = 32
HID = 128
NCONV = 8

NP = 10240
NW = 32
EW = 10240
NB = EW // 128
EP = NW * EW
ROWS_PER_TILE = NP // 16

_mesh = plsc.VectorSubcoreMesh(core_axis_name="c", subcore_axis_name="s")


def _dot(a, b):
    return lax.dot_general(a, b, (((1,), (0,)), ((), ())),
                           preferred_element_type=jnp.float32)


def _make_sc_gather(D):
    @functools.partial(
        pl.kernel,
        out_type=jax.ShapeDtypeStruct((EP, D), jnp.float32),
        mesh=_mesh,
        scratch_types=[
            pltpu.VMEM((NB, 128), jnp.int32),
            pltpu.VMEM((128, D), jnp.float32),
            pltpu.SemaphoreType.DMA,
        ],
    )
    def k(table, idx2, out, idx_v, rows_v, sem):
        wid = lax.axis_index("c") * 16 + lax.axis_index("s")
        pltpu.sync_copy(idx2.at[pl.ds(wid * NB, NB)], idx_v)

        @pl.loop(0, NB)
        def _(j):
            pltpu.async_copy(table.at[idx_v.at[j]], rows_v, sem).wait()
            pltpu.sync_copy(rows_v, out.at[pl.ds(wid * EW + j * 128, 128)])

    return k


def _make_sc_scatter_add(D):
    @functools.partial(
        pl.kernel,
        out_type=jax.ShapeDtypeStruct((2 * NP, D), jnp.float32),
        mesh=_mesh,
        scratch_types=[
            pltpu.VMEM((NB, 128), jnp.int32),
            pltpu.VMEM((128, D), jnp.float32),
            pltpu.VMEM_SHARED((NP, D), jnp.float32),
            pltpu.SemaphoreType.DMA,
        ],
    )
    def k(rows_hbm, dst2, zeros_hbm, out, dst_v, rows_v, acc, sem):
        c = lax.axis_index("c")
        s = lax.axis_index("s")
        wid = c * 16 + s

        @pl.loop(0, ROWS_PER_TILE // 128)
        def _(z):
            pltpu.sync_copy(zeros_hbm,
                            acc.at[pl.ds(s * ROWS_PER_TILE + z * 128, 128)])

        plsc.subcore_barrier()
        pltpu.sync_copy(dst2.at[pl.ds(wid * NB, NB)], dst_v)

        @pl.loop(0, NB)
        def _(j):
            pltpu.sync_copy(rows_hbm.at[pl.ds(wid * EW + j * 128, 128)], rows_v)
            pltpu.sync_copy(rows_v, acc.at[dst_v.at[j]], add=True)

        plsc.subcore_barrier()

        @pl.loop(0, ROWS_PER_TILE // 128)
        def _(z):
            r = s * ROWS_PER_TILE + z * 128
            pltpu.sync_copy(acc.at[pl.ds(r, 128)], out.at[pl.ds(c * NP + r, 128)])

    return k


def _make_sc_gather_scatter():
    D = HID

    @functools.partial(
        pl.kernel,
        out_type=jax.ShapeDtypeStruct((2 * NP, D), jnp.float32),
        mesh=_mesh,
        scratch_types=[
            pltpu.VMEM((NB, 128), jnp.int32),
            pltpu.VMEM((NB, 128), jnp.int32),
            pltpu.VMEM((128, D), jnp.float32),
            pltpu.VMEM_SHARED((NP, D), jnp.float32),
            pltpu.SemaphoreType.DMA,
        ],
    )
    def k(table, src2, dst2, zeros_hbm, out, src_v, dst_v, rows_v, acc, sem):
        c = lax.axis_index("c")
        s = lax.axis_index("s")
        wid = c * 16 + s

        @pl.loop(0, ROWS_PER_TILE // 128)
        def _(z):
            pltpu.sync_copy(zeros_hbm,
                            acc.at[pl.ds(s * ROWS_PER_TILE + z * 128, 128)])

        plsc.subcore_barrier()
        pltpu.sync_copy(src2.at[pl.ds(wid * NB, NB)], src_v)
        pltpu.sync_copy(dst2.at[pl.ds(wid * NB, NB)], dst_v)

        @pl.loop(0, NB)
        def _(j):
            pltpu.async_copy(table.at[src_v.at[j]], rows_v, sem).wait()
            pltpu.sync_copy(rows_v, acc.at[dst_v.at[j]], add=True)

        plsc.subcore_barrier()

        @pl.loop(0, ROWS_PER_TILE // 128)
        def _(z):
            r = s * ROWS_PER_TILE + z * 128
            pltpu.sync_copy(acc.at[pl.ds(r, 128)], out.at[pl.ds(c * NP + r, 128)])

    return k


_sc_gather128 = _make_sc_gather(HID)
_sc_scatter128 = _make_sc_scatter_add(HID)
_sc_gs = _make_sc_gather_scatter()


BE = 2048
BN = 1024


def _full(shape):
    return pl.BlockSpec(shape, lambda i: tuple(0 for _ in shape))


def _msgs_body(ea_ref, xs_ref, w1_ref, b1_ref, a_ref, bb_ref, out_ref):
    ea = ea_ref[...]
    h = jnp.maximum(_dot(ea, w1_ref[...]) + b1_ref[...], 0.0)
    xs = xs_ref[...]
    p = _dot(xs, a_ref[...])
    m = _dot(xs, bb_ref[...])
    for k in range(8):
        m = m + h[:, k:k + 1] * p[:, k * EMB:(k + 1) * EMB]
    out_ref[:, 0:EMB] = m
    one0 = jnp.where(
        lax.broadcasted_iota(jnp.int32, (BE, EMB), 1) == 0, 1.0, 0.0)
    out_ref[:, EMB:2 * EMB] = one0
    out_ref[:, 2 * EMB:] = jnp.zeros((BE, HID - 2 * EMB), jnp.float32)


def _tc_msgs(ea, xs, w1, b1, a, bb):
    return pl.pallas_call(
        _msgs_body,
        grid=(EP // BE,),
        in_specs=[
            pl.BlockSpec((BE, 3), lambda i: (i, 0)),
            pl.BlockSpec((BE, HID), lambda i: (i, 0)),
            _full((3, 8)), _full((1, 8)), _full((HID, 8 * EMB)), _full((HID, EMB)),
        ],
        out_specs=pl.BlockSpec((BE, HID), lambda i: (i, 0)),
        out_shape=jax.ShapeDtypeStruct((EP, HID), jnp.float32),
    )(ea, xs, w1, b1, a, bb)


def _node_body(p_ref, x_ref, rw_ref, nb_ref, o_ref, dv_ref):
    p = p_ref[...]
    agg = p[0, :, 0:EMB] + p[1, :, 0:EMB]
    deg = p[0, :, EMB:EMB + 1] + p[1, :, EMB:EMB + 1] + 1.0
    o_ref[...] = jnp.maximum(
        agg + _dot(x_ref[...], rw_ref[...]) + nb_ref[...], 0.0)
    dv_ref[...] = lax.rsqrt(deg)


def _tc_node(p, x128, rw, nb):
    return pl.pallas_call(
        _node_body,
        grid=(NP // BN,),
        in_specs=[
            pl.BlockSpec((2, BN, HID), lambda i: (0, i, 0)),
            pl.BlockSpec((BN, HID), lambda i: (i, 0)),
            _full((HID, EMB)), _full((1, EMB)),
        ],
        out_specs=[pl.BlockSpec((BN, EMB), lambda i: (i, 0)),
                   pl.BlockSpec((BN, 1), lambda i: (i, 0))],
        out_shape=[jax.ShapeDtypeStruct((NP, EMB), jnp.float32),
                   jax.ShapeDtypeStruct((NP, 1), jnp.float32)],
    )(p, x128, rw, nb)


def _pre_body(x_ref, w_ref, dv_ref, o_ref):
    o_ref[...] = dv_ref[...] * _dot(x_ref[...], w_ref[...])


def _tc_pre(x, w, dv):
    din = x.shape[1]
    return pl.pallas_call(
        _pre_body,
        grid=(NP // BN,),
        in_specs=[
            pl.BlockSpec((BN, din), lambda i: (i, 0)),
            _full((din, HID)),
            pl.BlockSpec((BN, 1), lambda i: (i, 0)),
        ],
        out_specs=pl.BlockSpec((BN, HID), lambda i: (i, 0)),
        out_shape=jax.ShapeDtypeStruct((NP, HID), jnp.float32),
    )(x, w, dv)


def _post_res_body(p_ref, g_ref, dv_ref, b_ref, xid_ref, o_ref):
    p = p_ref[...]
    h = dv_ref[...] * (p[0] + p[1] + g_ref[...]) + b_ref[...]
    o_ref[...] = jnp.maximum(h + xid_ref[...], 0.0)


def _post_body(p_ref, g_ref, dv_ref, b_ref, o_ref):
    p = p_ref[...]
    h = dv_ref[...] * (p[0] + p[1] + g_ref[...]) + b_ref[...]
    o_ref[...] = jnp.maximum(h, 0.0)


def _tc_post(p, g, dv, b, xid):
    body = _post_body if xid is None else _post_res_body
    ins = [p, g, dv, b] + ([] if xid is None else [xid])
    specs = [
        pl.BlockSpec((2, BN, HID), lambda i: (0, i, 0)),
        pl.BlockSpec((BN, HID), lambda i: (i, 0)),
        pl.BlockSpec((BN, 1), lambda i: (i, 0)),
        _full((1, HID)),
    ] + ([] if xid is None else [pl.BlockSpec((BN, HID), lambda i: (i, 0))])
    return pl.pallas_call(
        body,
        grid=(NP // BN,),
        in_specs=specs,
        out_specs=pl.BlockSpec((BN, HID), lambda i: (i, 0)),
        out_shape=jax.ShapeDtypeStruct((NP, HID), jnp.float32),
    )(*ins)


def _uv_body(x_ref, wa_ref, wb_ref, u_ref, v_ref):
    x = x_ref[...]
    u_ref[...] = _dot(x, wa_ref[...])
    v_ref[...] = _dot(x, wb_ref[...])


def _tc_uv(x, wa, wb):
    return pl.pallas_call(
        _uv_body,
        grid=(NP // BN,),
        in_specs=[
            pl.BlockSpec((BN, HID), lambda i: (i, 0)),
            _full((HID, HID)), _full((HID, HID)),
        ],
        out_specs=[pl.BlockSpec((BN, HID), lambda i: (i, 0))] * 2,
        out_shape=[jax.ShapeDtypeStruct((NP, HID), jnp.float32)] * 2,
    )(x, wa, wb)


def _head_body(us_ref, vd_ref, b1_ref, w2_ref, b2_ref, w3_ref, b3_ref, o_ref):
    t = jnp.maximum(us_ref[...] + vd_ref[...] + b1_ref[...], 0.0)
    t2 = jnp.maximum(_dot(t, w2_ref[...]) + b2_ref[...] + t, 0.0)
    o_ref[...] = _dot(t2, w3_ref[...]) + b3_ref[...]


def _tc_head(us, vd, b1, w2, b2, w3, b3):
    return pl.pallas_call(
        _head_body,
        grid=(EP // BE,),
        in_specs=[
            pl.BlockSpec((BE, HID), lambda i: (i, 0)),
            pl.BlockSpec((BE, HID), lambda i: (i, 0)),
            _full((1, HID)), _full((HID, HID)), _full((1, HID)),
            _full((HID, 1)), _full((1, 1)),
        ],
        out_specs=pl.BlockSpec((BE, 1), lambda i: (i, 0)),
        out_shape=jax.ShapeDtypeStruct((EP, 1), jnp.float32),
    )(us, vd, b1, w2, b2, w3, b3)


def kernel(x, edge_attr, enn_w1, enn_b1, enn_w2, enn_b2, root_w, nn_bias,
           conv_ws, conv_bs, mlp_ws, mlp_bs, edge_index):
    f32 = jnp.float32
    src = edge_index[0]
    dst = edge_index[1]

    pad = EP - E
    src2 = jnp.concatenate([src, jnp.zeros((pad,), jnp.int32)]).reshape(EP // 128, 128)
    dst2 = jnp.concatenate([dst, jnp.full((pad,), NP - 1, jnp.int32)]).reshape(EP // 128, 128)
    ea_p = jnp.concatenate([edge_attr, jnp.zeros((pad, 3), f32)])

    x128 = jnp.zeros((NP, HID), f32).at[:N, :IN_CH].set(x)
    z128 = jnp.zeros((128, HID), f32)

    a_mat = jnp.zeros((HID, 8 * EMB), f32).at[:IN_CH].set(
        enn_w2.reshape(8, IN_CH, EMB).transpose(1, 0, 2).reshape(IN_CH, 8 * EMB))
    b_mat = jnp.zeros((HID, EMB), f32).at[:IN_CH].set(enn_b2.reshape(IN_CH, EMB))
    rw = jnp.zeros((HID, EMB), f32).at[:IN_CH].set(root_w)

    xs = _sc_gather128(x128, src2)
    msgs = _tc_msgs(ea_p, xs, enn_w1, enn_b1.reshape(1, 8), a_mat, b_mat)
    pm = _sc_scatter128(msgs, dst2, z128).reshape(2, NP, HID)
    xc, dinv = _tc_node(pm, x128, rw, nn_bias.reshape(1, EMB))

    for i in range(NCONV):
        g = _tc_pre(xc, conv_ws[i], dinv)
        ps = _sc_gs(g, src2, dst2, z128).reshape(2, NP, HID)
        xc = _tc_post(ps, g, dinv, conv_bs[i].reshape(1, HID),
                      xc if i > 0 else None)

    u, v = _tc_uv(xc, mlp_ws[0][:HID], mlp_ws[0][HID:])
    us = _sc_gather128(u, src2)
    vd = _sc_gather128(v, dst2)
    out = _tc_head(us, vd, mlp_bs[0].reshape(1, HID), mlp_ws[1],
                   mlp_bs[1].reshape(1, HID), mlp_ws[2], mlp_bs[2].reshape(1, 1))
    return out[:E]


# speedup vs baseline: 3.7624x; 1.1121x over previous
import functools

import jax
import jax.numpy as jnp
from jax import lax
from jax.experimental import pallas as pl
from jax.experimental.pallas import tpu as pltpu
from jax.experimental.pallas import tpu_sc as plsc

N = 10000
E = 320000
IN_CH = 13
EMB = 32
HID = 128
NCONV = 8

NP = 10240
NW = 32
EW = 10240
NB = EW // 128
EP = NW * EW
ROWS_PER_TILE = NP // 16

_mesh = plsc.VectorSubcoreMesh(core_axis_name="c", subcore_axis_name="s")


def _dot(a, b):
    return lax.dot_general(a, b, (((1,), (0,)), ((), ())),
                           preferred_element_type=jnp.float32)


def _make_sc_gather(D):
    @functools.partial(
        pl.kernel,
        out_type=jax.ShapeDtypeStruct((EP, D), jnp.float32),
        mesh=_mesh,
        scratch_types=[
            pltpu.VMEM((NB, 128), jnp.int32),
            pltpu.VMEM((128, D), jnp.float32),
            pltpu.VMEM((128, D), jnp.float32),
            pltpu.SemaphoreType.DMA,
            pltpu.SemaphoreType.DMA,
        ],
    )
    def k(table, idx2, out, idx_v, b0, b1, s0, s1):
        wid = lax.axis_index("c") * 16 + lax.axis_index("s")
        base = wid * EW
        pltpu.sync_copy(idx2.at[pl.ds(wid * NB, NB)], idx_v)
        pltpu.async_copy(table.at[idx_v.at[0]], b0, s0)

        @pl.loop(0, NB, step=2)
        def _(j):
            pltpu.async_copy(table.at[idx_v.at[j + 1]], b1, s1)
            pltpu.make_async_copy(table.at[idx_v.at[0]], b0, s0).wait()
            pltpu.sync_copy(b0, out.at[pl.ds(base + j * 128, 128)])

            @pl.when(j + 2 < NB)
            def _():
                pltpu.async_copy(table.at[idx_v.at[j + 2]], b0, s0)

            pltpu.make_async_copy(table.at[idx_v.at[0]], b1, s1).wait()
            pltpu.sync_copy(b1, out.at[pl.ds(base + (j + 1) * 128, 128)])

    return k


def _make_sc_scatter_add(D):
    @functools.partial(
        pl.kernel,
        out_type=jax.ShapeDtypeStruct((2 * NP, D), jnp.float32),
        mesh=_mesh,
        scratch_types=[
            pltpu.VMEM((NB, 128), jnp.int32),
            pltpu.VMEM((128, D), jnp.float32),
            pltpu.VMEM((128, D), jnp.float32),
            pltpu.VMEM_SHARED((NP, D), jnp.float32),
            pltpu.SemaphoreType.DMA,
            pltpu.SemaphoreType.DMA,
        ],
    )
    def k(rows_hbm, dst2, zeros_hbm, out, dst_v, b0, b1, acc, s0, s1):
        c = lax.axis_index("c")
        s = lax.axis_index("s")
        wid = c * 16 + s

        @pl.loop(0, ROWS_PER_TILE // 128)
        def _(z):
            pltpu.sync_copy(zeros_hbm,
                            acc.at[pl.ds(s * ROWS_PER_TILE + z * 128, 128)])

        plsc.subcore_barrier()
        pltpu.sync_copy(dst2.at[pl.ds(wid * NB, NB)], dst_v)
        base = wid * EW
        pltpu.async_copy(rows_hbm.at[pl.ds(base, 128)], b0, s0)

        @pl.loop(0, NB, step=2)
        def _(j):
            pltpu.async_copy(rows_hbm.at[pl.ds(base + (j + 1) * 128, 128)], b1, s1)
            pltpu.make_async_copy(rows_hbm.at[pl.ds(0, 128)], b0, s0).wait()
            pltpu.sync_copy(b0, acc.at[dst_v.at[j]], add=True)

            @pl.when(j + 2 < NB)
            def _():
                pltpu.async_copy(rows_hbm.at[pl.ds(base + (j + 2) * 128, 128)], b0, s0)

            pltpu.make_async_copy(rows_hbm.at[pl.ds(0, 128)], b1, s1).wait()
            pltpu.sync_copy(b1, acc.at[dst_v.at[j + 1]], add=True)

        plsc.subcore_barrier()

        @pl.loop(0, ROWS_PER_TILE // 128)
        def _(z):
            r = s * ROWS_PER_TILE + z * 128
            pltpu.sync_copy(acc.at[pl.ds(r, 128)], out.at[pl.ds(c * NP + r, 128)])

    return k


def _make_sc_gather_scatter():
    D = HID

    @functools.partial(
        pl.kernel,
        out_type=jax.ShapeDtypeStruct((2 * NP, D), jnp.float32),
        mesh=_mesh,
        scratch_types=[
            pltpu.VMEM((NB // 2, 128), jnp.int32),
            pltpu.VMEM((NB // 2, 128), jnp.int32),
            pltpu.VMEM((128, D), jnp.float32),
            pltpu.VMEM((128, D), jnp.float32),
            pltpu.VMEM_SHARED((NP, D), jnp.float32),
            pltpu.SemaphoreType.DMA,
            pltpu.SemaphoreType.DMA,
        ],
    )
    def k(table, src2, dst2, zeros_hbm, out, src_v, dst_v, b0, b1, acc, s0, s1):
        c = lax.axis_index("c")
        s = lax.axis_index("s")
        wid = c * 16 + s
        nb2 = NB // 2

        @pl.loop(0, ROWS_PER_TILE // 128)
        def _(z):
            pltpu.sync_copy(zeros_hbm,
                            acc.at[pl.ds(s * ROWS_PER_TILE + z * 128, 128)])

        plsc.subcore_barrier()

        for ph in range(2):
            pltpu.sync_copy(src2.at[pl.ds(wid * NB + ph * nb2, nb2)], src_v)
            pltpu.sync_copy(dst2.at[pl.ds(wid * NB + ph * nb2, nb2)], dst_v)
            pltpu.async_copy(table.at[src_v.at[0]], b0, s0)

            @pl.loop(0, nb2, step=2)
            def _(j):
                pltpu.async_copy(table.at[src_v.at[j + 1]], b1, s1)
                pltpu.make_async_copy(table.at[src_v.at[0]], b0, s0).wait()
                pltpu.sync_copy(b0, acc.at[dst_v.at[j]], add=True)

                @pl.when(j + 2 < nb2)
                def _():
                    pltpu.async_copy(table.at[src_v.at[j + 2]], b0, s0)

                pltpu.make_async_copy(table.at[src_v.at[0]], b1, s1).wait()
                pltpu.sync_copy(b1, acc.at[dst_v.at[j + 1]], add=True)

        plsc.subcore_barrier()

        @pl.loop(0, ROWS_PER_TILE // 128)
        def _(z):
            r = s * ROWS_PER_TILE + z * 128
            pltpu.sync_copy(acc.at[pl.ds(r, 128)], out.at[pl.ds(c * NP + r, 128)])

    return k


_sc_gather128 = _make_sc_gather(HID)
_sc_scatter128 = _make_sc_scatter_add(HID)
_sc_gs = _make_sc_gather_scatter()


BE = 2048
BN = 1024


def _full(shape):
    return pl.BlockSpec(shape, lambda i: tuple(0 for _ in shape))


def _msgs_body(ea_ref, xs_ref, w1_ref, b1_ref, a_ref, bb_ref, out_ref):
    ea = ea_ref[...]
    h = jnp.maximum(_dot(ea, w1_ref[...]) + b1_ref[...], 0.0)
    xs = xs_ref[...]
    p = _dot(xs, a_ref[...])
    m = _dot(xs, bb_ref[...])
    for k in range(8):
        m = m + h[:, k:k + 1] * p[:, k * EMB:(k + 1) * EMB]
    out_ref[:, 0:EMB] = m
    one0 = jnp.where(
        lax.broadcasted_iota(jnp.int32, (BE, EMB), 1) == 0, 1.0, 0.0)
    out_ref[:, EMB:2 * EMB] = one0
    out_ref[:, 2 * EMB:] = jnp.zeros((BE, HID - 2 * EMB), jnp.float32)


def _tc_msgs(ea, xs, w1, b1, a, bb):
    return pl.pallas_call(
        _msgs_body,
        grid=(EP // BE,),
        in_specs=[
            pl.BlockSpec((BE, 3), lambda i: (i, 0)),
            pl.BlockSpec((BE, HID), lambda i: (i, 0)),
            _full((3, 8)), _full((1, 8)), _full((HID, 8 * EMB)), _full((HID, EMB)),
        ],
        out_specs=pl.BlockSpec((BE, HID), lambda i: (i, 0)),
        out_shape=jax.ShapeDtypeStruct((EP, HID), jnp.float32),
    )(ea, xs, w1, b1, a, bb)


def _node_body(p_ref, x_ref, rw_ref, nb_ref, o_ref, dv_ref):
    p = p_ref[...]
    agg = p[0, :, 0:EMB] + p[1, :, 0:EMB]
    deg = p[0, :, EMB:EMB + 1] + p[1, :, EMB:EMB + 1] + 1.0
    o_ref[...] = jnp.maximum(
        agg + _dot(x_ref[...], rw_ref[...]) + nb_ref[...], 0.0)
    dv_ref[...] = lax.rsqrt(deg)


def _tc_node(p, x128, rw, nb):
    return pl.pallas_call(
        _node_body,
        grid=(NP // BN,),
        in_specs=[
            pl.BlockSpec((2, BN, HID), lambda i: (0, i, 0)),
            pl.BlockSpec((BN, HID), lambda i: (i, 0)),
            _full((HID, EMB)), _full((1, EMB)),
        ],
        out_specs=[pl.BlockSpec((BN, EMB), lambda i: (i, 0)),
                   pl.BlockSpec((BN, 1), lambda i: (i, 0))],
        out_shape=[jax.ShapeDtypeStruct((NP, EMB), jnp.float32),
                   jax.ShapeDtypeStruct((NP, 1), jnp.float32)],
    )(p, x128, rw, nb)


def _pre_body(x_ref, w_ref, dv_ref, o_ref):
    o_ref[...] = dv_ref[...] * _dot(x_ref[...], w_ref[...])


def _tc_pre(x, w, dv):
    din = x.shape[1]
    return pl.pallas_call(
        _pre_body,
        grid=(NP // BN,),
        in_specs=[
            pl.BlockSpec((BN, din), lambda i: (i, 0)),
            _full((din, HID)),
            pl.BlockSpec((BN, 1), lambda i: (i, 0)),
        ],
        out_specs=pl.BlockSpec((BN, HID), lambda i: (i, 0)),
        out_shape=jax.ShapeDtypeStruct((NP, HID), jnp.float32),
    )(x, w, dv)


def _post_res_body(p_ref, g_ref, dv_ref, b_ref, xid_ref, o_ref):
    p = p_ref[...]
    h = dv_ref[...] * (p[0] + p[1] + g_ref[...]) + b_ref[...]
    o_ref[...] = jnp.maximum(h + xid_ref[...], 0.0)


def _post_body(p_ref, g_ref, dv_ref, b_ref, o_ref):
    p = p_ref[...]
    h = dv_ref[...] * (p[0] + p[1] + g_ref[...]) + b_ref[...]
    o_ref[...] = jnp.maximum(h, 0.0)


def _tc_post(p, g, dv, b, xid):
    body = _post_body if xid is None else _post_res_body
    ins = [p, g, dv, b] + ([] if xid is None else [xid])
    specs = [
        pl.BlockSpec((2, BN, HID), lambda i: (0, i, 0)),
        pl.BlockSpec((BN, HID), lambda i: (i, 0)),
        pl.BlockSpec((BN, 1), lambda i: (i, 0)),
        _full((1, HID)),
    ] + ([] if xid is None else [pl.BlockSpec((BN, HID), lambda i: (i, 0))])
    return pl.pallas_call(
        body,
        grid=(NP // BN,),
        in_specs=specs,
        out_specs=pl.BlockSpec((BN, HID), lambda i: (i, 0)),
        out_shape=jax.ShapeDtypeStruct((NP, HID), jnp.float32),
    )(*ins)


def _uv_body(x_ref, wa_ref, wb_ref, u_ref, v_ref):
    x = x_ref[...]
    u_ref[...] = _dot(x, wa_ref[...])
    v_ref[...] = _dot(x, wb_ref[...])


def _tc_uv(x, wa, wb):
    return pl.pallas_call(
        _uv_body,
        grid=(NP // BN,),
        in_specs=[
            pl.BlockSpec((BN, HID), lambda i: (i, 0)),
            _full((HID, HID)), _full((HID, HID)),
        ],
        out_specs=[pl.BlockSpec((BN, HID), lambda i: (i, 0))] * 2,
        out_shape=[jax.ShapeDtypeStruct((NP, HID), jnp.float32)] * 2,
    )(x, wa, wb)


def _head_body(us_ref, vd_ref, b1_ref, w2_ref, b2_ref, w3_ref, b3_ref, o_ref):
    t = jnp.maximum(us_ref[...] + vd_ref[...] + b1_ref[...], 0.0)
    t2 = jnp.maximum(_dot(t, w2_ref[...]) + b2_ref[...] + t, 0.0)
    o_ref[...] = _dot(t2, w3_ref[...]) + b3_ref[...]


def _tc_head(us, vd, b1, w2, b2, w3, b3):
    return pl.pallas_call(
        _head_body,
        grid=(EP // BE,),
        in_specs=[
            pl.BlockSpec((BE, HID), lambda i: (i, 0)),
            pl.BlockSpec((BE, HID), lambda i: (i, 0)),
            _full((1, HID)), _full((HID, HID)), _full((1, HID)),
            _full((HID, 1)), _full((1, 1)),
        ],
        out_specs=pl.BlockSpec((BE, 1), lambda i: (i, 0)),
        out_shape=jax.ShapeDtypeStruct((EP, 1), jnp.float32),
    )(us, vd, b1, w2, b2, w3, b3)


def kernel(x, edge_attr, enn_w1, enn_b1, enn_w2, enn_b2, root_w, nn_bias,
           conv_ws, conv_bs, mlp_ws, mlp_bs, edge_index):
    f32 = jnp.float32
    src = edge_index[0]
    dst = edge_index[1]

    pad = EP - E
    src2 = jnp.concatenate([src, jnp.zeros((pad,), jnp.int32)]).reshape(EP // 128, 128)
    dst2 = jnp.concatenate([dst, jnp.full((pad,), NP - 1, jnp.int32)]).reshape(EP // 128, 128)
    ea_p = jnp.concatenate([edge_attr, jnp.zeros((pad, 3), f32)])

    x128 = jnp.zeros((NP, HID), f32).at[:N, :IN_CH].set(x)
    z128 = jnp.zeros((128, HID), f32)

    a_mat = jnp.zeros((HID, 8 * EMB), f32).at[:IN_CH].set(
        enn_w2.reshape(8, IN_CH, EMB).transpose(1, 0, 2).reshape(IN_CH, 8 * EMB))
    b_mat = jnp.zeros((HID, EMB), f32).at[:IN_CH].set(enn_b2.reshape(IN_CH, EMB))
    rw = jnp.zeros((HID, EMB), f32).at[:IN_CH].set(root_w)

    xs = _sc_gather128(x128, src2)
    msgs = _tc_msgs(ea_p, xs, enn_w1, enn_b1.reshape(1, 8), a_mat, b_mat)
    pm = _sc_scatter128(msgs, dst2, z128).reshape(2, NP, HID)
    xc, dinv = _tc_node(pm, x128, rw, nn_bias.reshape(1, EMB))

    for i in range(NCONV):
        g = _tc_pre(xc, conv_ws[i], dinv)
        ps = _sc_gs(g, src2, dst2, z128).reshape(2, NP, HID)
        xc = _tc_post(ps, g, dinv, conv_bs[i].reshape(1, HID),
                      xc if i > 0 else None)

    u, v = _tc_uv(xc, mlp_ws[0][:HID], mlp_ws[0][HID:])
    us = _sc_gather128(u, src2)
    vd = _sc_gather128(v, dst2)
    out = _tc_head(us, vd, mlp_bs[0].reshape(1, HID), mlp_ws[1],
                   mlp_bs[1].reshape(1, HID), mlp_ws[2], mlp_bs[2].reshape(1, 1))
    return out[:E]


# speedup vs baseline: 3.7875x; 1.0067x over previous
import functools

import jax
import jax.numpy as jnp
from jax import lax
from jax.experimental import pallas as pl
from jax.experimental.pallas import tpu as pltpu
from jax.experimental.pallas import tpu_sc as plsc

N = 10000
E = 320000
IN_CH = 13
EMB = 32
HID = 128
NCONV = 8

NP = 10240
NW = 32
EW = 10240
NB = EW // 128
EP = NW * EW
ROWS_PER_TILE = NP // 16

_mesh = plsc.VectorSubcoreMesh(core_axis_name="c", subcore_axis_name="s")


def _dot(a, b):
    return lax.dot_general(a, b, (((1,), (0,)), ((), ())),
                           preferred_element_type=jnp.float32)


def _make_sc_gather(D):
    @functools.partial(
        pl.kernel,
        out_type=jax.ShapeDtypeStruct((EP, D), jnp.float32),
        mesh=_mesh,
        scratch_types=[
            pltpu.VMEM((NB, 128), jnp.int32),
            pltpu.VMEM((128, D), jnp.float32),
            pltpu.VMEM((128, D), jnp.float32),
            pltpu.SemaphoreType.DMA,
            pltpu.SemaphoreType.DMA,
        ],
    )
    def k(table, idx2, out, idx_v, b0, b1, s0, s1):
        wid = lax.axis_index("c") * 16 + lax.axis_index("s")
        base = wid * EW
        pltpu.sync_copy(idx2.at[pl.ds(wid * NB, NB)], idx_v)
        pltpu.async_copy(table.at[idx_v.at[0]], b0, s0)

        @pl.loop(0, NB, step=2)
        def _(j):
            pltpu.async_copy(table.at[idx_v.at[j + 1]], b1, s1)
            pltpu.make_async_copy(table.at[idx_v.at[0]], b0, s0).wait()
            pltpu.sync_copy(b0, out.at[pl.ds(base + j * 128, 128)])

            @pl.when(j + 2 < NB)
            def _():
                pltpu.async_copy(table.at[idx_v.at[j + 2]], b0, s0)

            pltpu.make_async_copy(table.at[idx_v.at[0]], b1, s1).wait()
            pltpu.sync_copy(b1, out.at[pl.ds(base + (j + 1) * 128, 128)])

    return k


def _make_sc_gather_add(D):
    @functools.partial(
        pl.kernel,
        out_type=jax.ShapeDtypeStruct((EP, D), jnp.float32),
        mesh=_mesh,
        scratch_types=[
            pltpu.VMEM((NB, 128), jnp.int32),
            pltpu.VMEM((NB, 128), jnp.int32),
            pltpu.VMEM((128, D), jnp.float32),
            pltpu.VMEM((128, D), jnp.float32),
            pltpu.SemaphoreType.DMA,
            pltpu.SemaphoreType.DMA,
        ],
    )
    def k(tu, tv, src2, dst2, out, src_v, dst_v, b0, b1, s0, s1):
        wid = lax.axis_index("c") * 16 + lax.axis_index("s")
        base = wid * EW
        pltpu.sync_copy(src2.at[pl.ds(wid * NB, NB)], src_v)
        pltpu.sync_copy(dst2.at[pl.ds(wid * NB, NB)], dst_v)
        pltpu.async_copy(tu.at[src_v.at[0]], b0, s0)

        @pl.loop(0, NB, step=2)
        def _(j):
            pltpu.async_copy(tu.at[src_v.at[j + 1]], b1, s1)
            pltpu.make_async_copy(tu.at[src_v.at[0]], b0, s0).wait()
            pltpu.async_copy(tv.at[dst_v.at[j]], b0, s0, add=True)
            pltpu.make_async_copy(tu.at[src_v.at[0]], b0, s0).wait()
            pltpu.sync_copy(b0, out.at[pl.ds(base + j * 128, 128)])

            @pl.when(j + 2 < NB)
            def _():
                pltpu.async_copy(tu.at[src_v.at[j + 2]], b0, s0)

            pltpu.make_async_copy(tu.at[src_v.at[0]], b1, s1).wait()
            pltpu.async_copy(tv.at[dst_v.at[j + 1]], b1, s1, add=True)
            pltpu.make_async_copy(tu.at[src_v.at[0]], b1, s1).wait()
            pltpu.sync_copy(b1, out.at[pl.ds(base + (j + 1) * 128, 128)])

    return k


def _make_sc_scatter_add(D):
    @functools.partial(
        pl.kernel,
        out_type=jax.ShapeDtypeStruct((2 * NP, D), jnp.float32),
        mesh=_mesh,
        scratch_types=[
            pltpu.VMEM((NB, 128), jnp.int32),
            pltpu.VMEM((128, D), jnp.float32),
            pltpu.VMEM((128, D), jnp.float32),
            pltpu.VMEM_SHARED((NP, D), jnp.float32),
            pltpu.SemaphoreType.DMA,
            pltpu.SemaphoreType.DMA,
        ],
    )
    def k(rows_hbm, dst2, zeros_hbm, out, dst_v, b0, b1, acc, s0, s1):
        c = lax.axis_index("c")
        s = lax.axis_index("s")
        wid = c * 16 + s

        @pl.loop(0, ROWS_PER_TILE // 128)
        def _(z):
            pltpu.sync_copy(zeros_hbm,
                            acc.at[pl.ds(s * ROWS_PER_TILE + z * 128, 128)])

        plsc.subcore_barrier()
        pltpu.sync_copy(dst2.at[pl.ds(wid * NB, NB)], dst_v)
        base = wid * EW
        pltpu.async_copy(rows_hbm.at[pl.ds(base, 128)], b0, s0)

        @pl.loop(0, NB, step=2)
        def _(j):
            pltpu.async_copy(rows_hbm.at[pl.ds(base + (j + 1) * 128, 128)], b1, s1)
            pltpu.make_async_copy(rows_hbm.at[pl.ds(0, 128)], b0, s0).wait()
            pltpu.sync_copy(b0, acc.at[dst_v.at[j]], add=True)

            @pl.when(j + 2 < NB)
            def _():
                pltpu.async_copy(rows_hbm.at[pl.ds(base + (j + 2) * 128, 128)], b0, s0)

            pltpu.make_async_copy(rows_hbm.at[pl.ds(0, 128)], b1, s1).wait()
            pltpu.sync_copy(b1, acc.at[dst_v.at[j + 1]], add=True)

        plsc.subcore_barrier()

        @pl.loop(0, ROWS_PER_TILE // 128)
        def _(z):
            r = s * ROWS_PER_TILE + z * 128
            pltpu.sync_copy(acc.at[pl.ds(r, 128)], out.at[pl.ds(c * NP + r, 128)])

    return k


def _make_sc_gather_scatter():
    D = HID

    @functools.partial(
        pl.kernel,
        out_type=jax.ShapeDtypeStruct((2 * NP, D), jnp.float32),
        mesh=_mesh,
        scratch_types=[
            pltpu.VMEM((NB // 2, 128), jnp.int32),
            pltpu.VMEM((NB // 2, 128), jnp.int32),
            pltpu.VMEM((128, D), jnp.float32),
            pltpu.VMEM((128, D), jnp.float32),
            pltpu.VMEM_SHARED((NP, D), jnp.float32),
            pltpu.SemaphoreType.DMA,
            pltpu.SemaphoreType.DMA,
        ],
    )
    def k(table, src2, dst2, zeros_hbm, out, src_v, dst_v, b0, b1, acc, s0, s1):
        c = lax.axis_index("c")
        s = lax.axis_index("s")
        wid = c * 16 + s
        nb2 = NB // 2

        @pl.loop(0, ROWS_PER_TILE // 128)
        def _(z):
            pltpu.sync_copy(zeros_hbm,
                            acc.at[pl.ds(s * ROWS_PER_TILE + z * 128, 128)])

        plsc.subcore_barrier()

        for ph in range(2):
            pltpu.sync_copy(src2.at[pl.ds(wid * NB + ph * nb2, nb2)], src_v)
            pltpu.sync_copy(dst2.at[pl.ds(wid * NB + ph * nb2, nb2)], dst_v)
            pltpu.async_copy(table.at[src_v.at[0]], b0, s0)

            @pl.loop(0, nb2, step=2)
            def _(j):
                pltpu.async_copy(table.at[src_v.at[j + 1]], b1, s1)
                pltpu.make_async_copy(table.at[src_v.at[0]], b0, s0).wait()
                pltpu.sync_copy(b0, acc.at[dst_v.at[j]], add=True)

                @pl.when(j + 2 < nb2)
                def _():
                    pltpu.async_copy(table.at[src_v.at[j + 2]], b0, s0)

                pltpu.make_async_copy(table.at[src_v.at[0]], b1, s1).wait()
                pltpu.sync_copy(b1, acc.at[dst_v.at[j + 1]], add=True)

        plsc.subcore_barrier()

        @pl.loop(0, ROWS_PER_TILE // 128)
        def _(z):
            r = s * ROWS_PER_TILE + z * 128
            pltpu.sync_copy(acc.at[pl.ds(r, 128)], out.at[pl.ds(c * NP + r, 128)])

    return k


_sc_gather128 = _make_sc_gather(HID)
_sc_gather_add128 = _make_sc_gather_add(HID)
_sc_scatter128 = _make_sc_scatter_add(HID)
_sc_gs = _make_sc_gather_scatter()


BE = 2048
BN = 1024


def _full(shape):
    return pl.BlockSpec(shape, lambda i: tuple(0 for _ in shape))


def _msgs_body(ea_ref, xs_ref, w1_ref, b1_ref, a_ref, bb_ref, out_ref):
    ea = ea_ref[...]
    h = jnp.maximum(_dot(ea, w1_ref[...]) + b1_ref[...], 0.0)
    xs = xs_ref[...]
    p = _dot(xs, a_ref[...])
    m = _dot(xs, bb_ref[...])
    for k in range(8):
        m = m + h[:, k:k + 1] * p[:, k * EMB:(k + 1) * EMB]
    out_ref[:, 0:EMB] = m
    one0 = jnp.where(
        lax.broadcasted_iota(jnp.int32, (BE, EMB), 1) == 0, 1.0, 0.0)
    out_ref[:, EMB:2 * EMB] = one0
    out_ref[:, 2 * EMB:] = jnp.zeros((BE, HID - 2 * EMB), jnp.float32)


def _tc_msgs(ea, xs, w1, b1, a, bb):
    return pl.pallas_call(
        _msgs_body,
        grid=(EP // BE,),
        in_specs=[
            pl.BlockSpec((BE, 3), lambda i: (i, 0)),
            pl.BlockSpec((BE, HID), lambda i: (i, 0)),
            _full((3, 8)), _full((1, 8)), _full((HID, 8 * EMB)), _full((HID, EMB)),
        ],
        out_specs=pl.BlockSpec((BE, HID), lambda i: (i, 0)),
        out_shape=jax.ShapeDtypeStruct((EP, HID), jnp.float32),
    )(ea, xs, w1, b1, a, bb)


def _node_body(p_ref, x_ref, rw_ref, nb_ref, o_ref, dv_ref):
    p = p_ref[...]
    agg = p[0, :, 0:EMB] + p[1, :, 0:EMB]
    deg = p[0, :, EMB:EMB + 1] + p[1, :, EMB:EMB + 1] + 1.0
    o_ref[...] = jnp.maximum(
        agg + _dot(x_ref[...], rw_ref[...]) + nb_ref[...], 0.0)
    dv_ref[...] = lax.rsqrt(deg)


def _tc_node(p, x128, rw, nb):
    return pl.pallas_call(
        _node_body,
        grid=(NP // BN,),
        in_specs=[
            pl.BlockSpec((2, BN, HID), lambda i: (0, i, 0)),
            pl.BlockSpec((BN, HID), lambda i: (i, 0)),
            _full((HID, EMB)), _full((1, EMB)),
        ],
        out_specs=[pl.BlockSpec((BN, EMB), lambda i: (i, 0)),
                   pl.BlockSpec((BN, 1), lambda i: (i, 0))],
        out_shape=[jax.ShapeDtypeStruct((NP, EMB), jnp.float32),
                   jax.ShapeDtypeStruct((NP, 1), jnp.float32)],
    )(p, x128, rw, nb)


def _pre_body(x_ref, w_ref, dv_ref, o_ref):
    o_ref[...] = dv_ref[...] * _dot(x_ref[...], w_ref[...])


def _tc_pre(x, w, dv):
    din = x.shape[1]
    return pl.pallas_call(
        _pre_body,
        grid=(NP // BN,),
        in_specs=[
            pl.BlockSpec((BN, din), lambda i: (i, 0)),
            _full((din, HID)),
            pl.BlockSpec((BN, 1), lambda i: (i, 0)),
        ],
        out_specs=pl.BlockSpec((BN, HID), lambda i: (i, 0)),
        out_shape=jax.ShapeDtypeStruct((NP, HID), jnp.float32),
    )(x, w, dv)


def _post_res_body(p_ref, g_ref, dv_ref, b_ref, xid_ref, o_ref):
    p = p_ref[...]
    h = dv_ref[...] * (p[0] + p[1] + g_ref[...]) + b_ref[...]
    o_ref[...] = jnp.maximum(h + xid_ref[...], 0.0)


def _post_body(p_ref, g_ref, dv_ref, b_ref, o_ref):
    p = p_ref[...]
    h = dv_ref[...] * (p[0] + p[1] + g_ref[...]) + b_ref[...]
    o_ref[...] = jnp.maximum(h, 0.0)


def _tc_post(p, g, dv, b, xid):
    body = _post_body if xid is None else _post_res_body
    ins = [p, g, dv, b] + ([] if xid is None else [xid])
    specs = [
        pl.BlockSpec((2, BN, HID), lambda i: (0, i, 0)),
        pl.BlockSpec((BN, HID), lambda i: (i, 0)),
        pl.BlockSpec((BN, 1), lambda i: (i, 0)),
        _full((1, HID)),
    ] + ([] if xid is None else [pl.BlockSpec((BN, HID), lambda i: (i, 0))])
    return pl.pallas_call(
        body,
        grid=(NP // BN,),
        in_specs=specs,
        out_specs=pl.BlockSpec((BN, HID), lambda i: (i, 0)),
        out_shape=jax.ShapeDtypeStruct((NP, HID), jnp.float32),
    )(*ins)


def _uv_body(x_ref, wa_ref, wb_ref, u_ref, v_ref):
    x = x_ref[...]
    u_ref[...] = _dot(x, wa_ref[...])
    v_ref[...] = _dot(x, wb_ref[...])


def _tc_uv(x, wa, wb):
    return pl.pallas_call(
        _uv_body,
        grid=(NP // BN,),
        in_specs=[
            pl.BlockSpec((BN, HID), lambda i: (i, 0)),
            _full((HID, HID)), _full((HID, HID)),
        ],
        out_specs=[pl.BlockSpec((BN, HID), lambda i: (i, 0))] * 2,
        out_shape=[jax.ShapeDtypeStruct((NP, HID), jnp.float32)] * 2,
    )(x, wa, wb)


def _head_body(uv_ref, b1_ref, w2_ref, b2_ref, w3_ref, b3_ref, o_ref):
    t = jnp.maximum(uv_ref[...] + b1_ref[...], 0.0)
    t2 = jnp.maximum(_dot(t, w2_ref[...]) + b2_ref[...] + t, 0.0)
    o_ref[...] = _dot(t2, w3_ref[...]) + b3_ref[...]


def _tc_head(uv, b1, w2, b2, w3, b3):
    return pl.pallas_call(
        _head_body,
        grid=(EP // BE,),
        in_specs=[
            pl.BlockSpec((BE, HID), lambda i: (i, 0)),
            _full((1, HID)), _full((HID, HID)), _full((1, HID)),
            _full((HID, 1)), _full((1, 1)),
        ],
        out_specs=pl.BlockSpec((BE, 1), lambda i: (i, 0)),
        out_shape=jax.ShapeDtypeStruct((EP, 1), jnp.float32),
    )(uv, b1, w2, b2, w3, b3)


def kernel(x, edge_attr, enn_w1, enn_b1, enn_w2, enn_b2, root_w, nn_bias,
           conv_ws, conv_bs, mlp_ws, mlp_bs, edge_index):
    f32 = jnp.float32
    src = edge_index[0]
    dst = edge_index[1]

    pad = EP - E
    src2 = jnp.concatenate([src, jnp.zeros((pad,), jnp.int32)]).reshape(EP // 128, 128)
    dst2 = jnp.concatenate([dst, jnp.full((pad,), NP - 1, jnp.int32)]).reshape(EP // 128, 128)
    ea_p = jnp.concatenate([edge_attr, jnp.zeros((pad, 3), f32)])

    x128 = jnp.zeros((NP, HID), f32).at[:N, :IN_CH].set(x)
    z128 = jnp.zeros((128, HID), f32)

    a_mat = jnp.zeros((HID, 8 * EMB), f32).at[:IN_CH].set(
        enn_w2.reshape(8, IN_CH, EMB).transpose(1, 0, 2).reshape(IN_CH, 8 * EMB))
    b_mat = jnp.zeros((HID, EMB), f32).at[:IN_CH].set(enn_b2.reshape(IN_CH, EMB))
    rw = jnp.zeros((HID, EMB), f32).at[:IN_CH].set(root_w)

    xs = _sc_gather128(x128, src2)
    msgs = _tc_msgs(ea_p, xs, enn_w1, enn_b1.reshape(1, 8), a_mat, b_mat)
    pm = _sc_scatter128(msgs, dst2, z128).reshape(2, NP, HID)
    xc, dinv = _tc_node(pm, x128, rw, nn_bias.reshape(1, EMB))

    for i in range(NCONV):
        g = _tc_pre(xc, conv_ws[i], dinv)
        ps = _sc_gs(g, src2, dst2, z128).reshape(2, NP, HID)
        xc = _tc_post(ps, g, dinv, conv_bs[i].reshape(1, HID),
                      xc if i > 0 else None)

    u, v = _tc_uv(xc, mlp_ws[0][:HID], mlp_ws[0][HID:])
    uv = _sc_gather_add128(u, v, src2, dst2)
    out = _tc_head(uv, mlp_bs[0].reshape(1, HID), mlp_ws[1],
                   mlp_bs[1].reshape(1, HID), mlp_ws[2], mlp_bs[2].reshape(1, 1))
    return out[:E]


# speedup vs baseline: 4.1184x; 1.0874x over previous
import functools

import jax
import jax.numpy as jnp
from jax import lax
from jax.experimental import pallas as pl
from jax.experimental.pallas import tpu as pltpu
from jax.experimental.pallas import tpu_sc as plsc

N = 10000
E = 320000
IN_CH = 13
EMB = 32
HID = 128
NCONV = 8

NP = 10240
NW = 32
EW = 10240
NB = EW // 128
EP = NW * EW
ROWS_PER_TILE = NP // 16

_mesh = plsc.VectorSubcoreMesh(core_axis_name="c", subcore_axis_name="s")


def _dot(a, b):
    return lax.dot_general(a, b, (((1,), (0,)), ((), ())),
                           preferred_element_type=jnp.float32)


def _make_sc_gather(D):
    @functools.partial(
        pl.kernel,
        out_type=jax.ShapeDtypeStruct((EP, D), jnp.float32),
        mesh=_mesh,
        scratch_types=[
            pltpu.VMEM((NB, 128), jnp.int32),
            pltpu.VMEM((128, D), jnp.float32),
            pltpu.VMEM((128, D), jnp.float32),
            pltpu.SemaphoreType.DMA,
            pltpu.SemaphoreType.DMA,
        ],
    )
    def k(table, idx2, out, idx_v, b0, b1, s0, s1):
        wid = lax.axis_index("c") * 16 + lax.axis_index("s")
        base = wid * EW
        pltpu.sync_copy(idx2.at[pl.ds(wid * NB, NB)], idx_v)
        pltpu.async_copy(table.at[idx_v.at[0]], b0, s0)

        @pl.loop(0, NB, step=2)
        def _(j):
            pltpu.async_copy(table.at[idx_v.at[j + 1]], b1, s1)
            pltpu.make_async_copy(table.at[idx_v.at[0]], b0, s0).wait()
            pltpu.sync_copy(b0, out.at[pl.ds(base + j * 128, 128)])

            @pl.when(j + 2 < NB)
            def _():
                pltpu.async_copy(table.at[idx_v.at[j + 2]], b0, s0)

            pltpu.make_async_copy(table.at[idx_v.at[0]], b1, s1).wait()
            pltpu.sync_copy(b1, out.at[pl.ds(base + (j + 1) * 128, 128)])

    return k


def _make_sc_gather_add(D):
    @functools.partial(
        pl.kernel,
        out_type=jax.ShapeDtypeStruct((EP, D), jnp.float32),
        mesh=_mesh,
        scratch_types=[
            pltpu.VMEM((NB, 128), jnp.int32),
            pltpu.VMEM((NB, 128), jnp.int32),
            pltpu.VMEM((128, D), jnp.float32),
            pltpu.VMEM((128, D), jnp.float32),
            pltpu.SemaphoreType.DMA,
            pltpu.SemaphoreType.DMA,
        ],
    )
    def k(tu, tv, src2, dst2, out, src_v, dst_v, b0, b1, s0, s1):
        wid = lax.axis_index("c") * 16 + lax.axis_index("s")
        base = wid * EW
        pltpu.sync_copy(src2.at[pl.ds(wid * NB, NB)], src_v)
        pltpu.sync_copy(dst2.at[pl.ds(wid * NB, NB)], dst_v)
        pltpu.async_copy(tu.at[src_v.at[0]], b0, s0)

        @pl.loop(0, NB, step=2)
        def _(j):
            pltpu.async_copy(tu.at[src_v.at[j + 1]], b1, s1)
            pltpu.make_async_copy(tu.at[src_v.at[0]], b0, s0).wait()
            pltpu.async_copy(tv.at[dst_v.at[j]], b0, s0, add=True)
            pltpu.make_async_copy(tu.at[src_v.at[0]], b0, s0).wait()
            pltpu.sync_copy(b0, out.at[pl.ds(base + j * 128, 128)])

            @pl.when(j + 2 < NB)
            def _():
                pltpu.async_copy(tu.at[src_v.at[j + 2]], b0, s0)

            pltpu.make_async_copy(tu.at[src_v.at[0]], b1, s1).wait()
            pltpu.async_copy(tv.at[dst_v.at[j + 1]], b1, s1, add=True)
            pltpu.make_async_copy(tu.at[src_v.at[0]], b1, s1).wait()
            pltpu.sync_copy(b1, out.at[pl.ds(base + (j + 1) * 128, 128)])

    return k


def _make_sc_scatter_add(D):
    @functools.partial(
        pl.kernel,
        out_type=jax.ShapeDtypeStruct((2 * NP, D), jnp.float32),
        mesh=_mesh,
        scratch_types=[
            pltpu.VMEM((NB, 128), jnp.int32),
            pltpu.VMEM((128, D), jnp.float32),
            pltpu.VMEM((128, D), jnp.float32),
            pltpu.VMEM_SHARED((NP, D), jnp.float32),
            pltpu.SemaphoreType.DMA,
            pltpu.SemaphoreType.DMA,
        ],
    )
    def k(rows_hbm, dst2, zeros_hbm, out, dst_v, b0, b1, acc, s0, s1):
        c = lax.axis_index("c")
        s = lax.axis_index("s")
        wid = c * 16 + s

        @pl.loop(0, ROWS_PER_TILE // 128)
        def _(z):
            pltpu.sync_copy(zeros_hbm,
                            acc.at[pl.ds(s * ROWS_PER_TILE + z * 128, 128)])

        plsc.subcore_barrier()
        plsc.subcore_barrier()
        pltpu.sync_copy(dst2.at[pl.ds(wid * NB, NB)], dst_v)
        base = wid * EW
        pltpu.async_copy(rows_hbm.at[pl.ds(base, 128)], b0, s0)

        @pl.loop(0, NB, step=2)
        def _(j):
            pltpu.async_copy(rows_hbm.at[pl.ds(base + (j + 1) * 128, 128)], b1, s1)
            pltpu.make_async_copy(rows_hbm.at[pl.ds(0, 128)], b0, s0).wait()
            pltpu.sync_copy(b0, acc.at[dst_v.at[j]], add=True)

            @pl.when(j + 2 < NB)
            def _():
                pltpu.async_copy(rows_hbm.at[pl.ds(base + (j + 2) * 128, 128)], b0, s0)

            pltpu.make_async_copy(rows_hbm.at[pl.ds(0, 128)], b1, s1).wait()
            pltpu.sync_copy(b1, acc.at[dst_v.at[j + 1]], add=True)

        plsc.subcore_barrier()
        plsc.subcore_barrier()
        plsc.subcore_barrier()

        @pl.loop(0, ROWS_PER_TILE // 128)
        def _(z):
            r = s * ROWS_PER_TILE + z * 128
            pltpu.sync_copy(acc.at[pl.ds(r, 128)], out.at[pl.ds(c * NP + r, 128)])

    return k


def _make_sc_gather_scatter():
    D = HID

    @functools.partial(
        pl.kernel,
        out_type=jax.ShapeDtypeStruct((2 * NP, D), jnp.float32),
        mesh=_mesh,
        scratch_types=[
            pltpu.VMEM((NB // 2, 128), jnp.int32),
            pltpu.VMEM((NB // 2, 128), jnp.int32),
            pltpu.VMEM((128, D), jnp.float32),
            pltpu.VMEM((128, D), jnp.float32),
            pltpu.VMEM_SHARED((NP, D), jnp.float32),
            pltpu.SemaphoreType.DMA,
            pltpu.SemaphoreType.DMA,
        ],
    )
    def k(table, src2, dst2, zeros_hbm, out, src_v, dst_v, b0, b1, acc, s0, s1):
        c = lax.axis_index("c")
        s = lax.axis_index("s")
        wid = c * 16 + s
        nb2 = NB // 2

        @pl.loop(0, ROWS_PER_TILE // 128)
        def _(z):
            pltpu.sync_copy(zeros_hbm,
                            acc.at[pl.ds(s * ROWS_PER_TILE + z * 128, 128)])

        plsc.subcore_barrier()
        plsc.subcore_barrier()

        for ph in range(2):
            pltpu.sync_copy(src2.at[pl.ds(wid * NB + ph * nb2, nb2)], src_v)
            pltpu.sync_copy(dst2.at[pl.ds(wid * NB + ph * nb2, nb2)], dst_v)
            pltpu.async_copy(table.at[src_v.at[0]], b0, s0)

            @pl.loop(0, nb2, step=2)
            def _(j):
                pltpu.async_copy(table.at[src_v.at[j + 1]], b1, s1)
                pltpu.make_async_copy(table.at[src_v.at[0]], b0, s0).wait()
                pltpu.sync_copy(b0, acc.at[dst_v.at[j]], add=True)

                @pl.when(j + 2 < nb2)
                def _():
                    pltpu.async_copy(table.at[src_v.at[j + 2]], b0, s0)

                pltpu.make_async_copy(table.at[src_v.at[0]], b1, s1).wait()
                pltpu.sync_copy(b1, acc.at[dst_v.at[j + 1]], add=True)

        plsc.subcore_barrier()
        plsc.subcore_barrier()
        plsc.subcore_barrier()

        @pl.loop(0, ROWS_PER_TILE // 128)
        def _(z):
            r = s * ROWS_PER_TILE + z * 128
            pltpu.sync_copy(acc.at[pl.ds(r, 128)], out.at[pl.ds(c * NP + r, 128)])

    return k


_sc_gather128 = _make_sc_gather(HID)
_sc_gather_add128 = _make_sc_gather_add(HID)
_sc_scatter128 = _make_sc_scatter_add(HID)
_sc_gs = _make_sc_gather_scatter()


BE = 2048
BN = 1024


def _full(shape):
    return pl.BlockSpec(shape, lambda i: tuple(0 for _ in shape))


def _msgs_body(ea_ref, xs_ref, w1_ref, b1_ref, w2_ref, b2_ref, out_ref):
    ea = ea_ref[...]
    h = jnp.maximum(_dot(ea, w1_ref[...]) + b1_ref[...], 0.0)
    theta = _dot(h, w2_ref[...]) + b2_ref[...]
    bf = jnp.bfloat16
    xs = xs_ref[...].astype(bf).astype(jnp.float32)
    theta = theta.astype(bf).astype(jnp.float32)
    m = xs[:, 0:1] * theta[:, 0:EMB]
    for i in range(1, IN_CH):
        m = m + xs[:, i:i + 1] * theta[:, i * EMB:(i + 1) * EMB]
    out_ref[:, 0:EMB] = m
    one0 = jnp.where(
        lax.broadcasted_iota(jnp.int32, (BE, EMB), 1) == 0, 1.0, 0.0)
    out_ref[:, EMB:2 * EMB] = one0
    out_ref[:, 2 * EMB:] = jnp.zeros((BE, HID - 2 * EMB), jnp.float32)


def _tc_msgs(ea, xs, w1, b1, w2, b2):
    return pl.pallas_call(
        _msgs_body,
        grid=(EP // BE,),
        in_specs=[
            pl.BlockSpec((BE, 3), lambda i: (i, 0)),
            pl.BlockSpec((BE, HID), lambda i: (i, 0)),
            _full((3, 8)), _full((1, 8)), _full((8, IN_CH * EMB)),
            _full((1, IN_CH * EMB)),
        ],
        out_specs=pl.BlockSpec((BE, HID), lambda i: (i, 0)),
        out_shape=jax.ShapeDtypeStruct((EP, HID), jnp.float32),
    )(ea, xs, w1, b1, w2, b2)


def _node_body(p_ref, x_ref, rw_ref, nb_ref, w0_ref, dv_ref, o_ref, g_ref):
    p = p_ref[...]
    agg = p[0, :, 0:EMB] + p[1, :, 0:EMB]
    x1 = jnp.maximum(agg + _dot(x_ref[...], rw_ref[...]) + nb_ref[...], 0.0)
    o_ref[...] = x1
    g_ref[...] = dv_ref[...] * _dot(x1, w0_ref[...])


def _tc_node(p, x128, rw, nb, w0, dv):
    return pl.pallas_call(
        _node_body,
        grid=(NP // BN,),
        in_specs=[
            pl.BlockSpec((2, BN, HID), lambda i: (0, i, 0)),
            pl.BlockSpec((BN, HID), lambda i: (i, 0)),
            _full((HID, EMB)), _full((1, EMB)), _full((EMB, HID)),
            pl.BlockSpec((BN, 1), lambda i: (i, 0)),
        ],
        out_specs=[pl.BlockSpec((BN, EMB), lambda i: (i, 0)),
                   pl.BlockSpec((BN, HID), lambda i: (i, 0))],
        out_shape=[jax.ShapeDtypeStruct((NP, EMB), jnp.float32),
                   jax.ShapeDtypeStruct((NP, HID), jnp.float32)],
    )(p, x128, rw, nb, w0, dv)


def _post_pre_res_body(p_ref, g_ref, dv_ref, b_ref, w_ref, xid_ref,
                       o_ref, gn_ref):
    p = p_ref[...]
    dv = dv_ref[...]
    h = dv * (p[0] + p[1] + g_ref[...]) + b_ref[...]
    x = jnp.maximum(h + xid_ref[...], 0.0)
    o_ref[...] = x
    gn_ref[...] = dv * _dot(x, w_ref[...])


def _post_pre_body(p_ref, g_ref, dv_ref, b_ref, w_ref, o_ref, gn_ref):
    p = p_ref[...]
    dv = dv_ref[...]
    h = dv * (p[0] + p[1] + g_ref[...]) + b_ref[...]
    x = jnp.maximum(h, 0.0)
    o_ref[...] = x
    gn_ref[...] = dv * _dot(x, w_ref[...])


def _tc_post_pre(p, g, dv, b, w_next, xid):
    body = _post_pre_body if xid is None else _post_pre_res_body
    ins = [p, g, dv, b, w_next] + ([] if xid is None else [xid])
    specs = [
        pl.BlockSpec((2, BN, HID), lambda i: (0, i, 0)),
        pl.BlockSpec((BN, HID), lambda i: (i, 0)),
        pl.BlockSpec((BN, 1), lambda i: (i, 0)),
        _full((1, HID)), _full((HID, HID)),
    ] + ([] if xid is None else [pl.BlockSpec((BN, HID), lambda i: (i, 0))])
    return pl.pallas_call(
        body,
        grid=(NP // BN,),
        in_specs=specs,
        out_specs=[pl.BlockSpec((BN, HID), lambda i: (i, 0))] * 2,
        out_shape=[jax.ShapeDtypeStruct((NP, HID), jnp.float32)] * 2,
    )(*ins)


def _post_uv_body(p_ref, g_ref, dv_ref, b_ref, xid_ref, wa_ref, wb_ref,
                  u_ref, v_ref):
    p = p_ref[...]
    h = dv_ref[...] * (p[0] + p[1] + g_ref[...]) + b_ref[...]
    x = jnp.maximum(h + xid_ref[...], 0.0)
    u_ref[...] = _dot(x, wa_ref[...])
    v_ref[...] = _dot(x, wb_ref[...])


def _tc_post_uv(p, g, dv, b, xid, wa, wb):
    return pl.pallas_call(
        _post_uv_body,
        grid=(NP // BN,),
        in_specs=[
            pl.BlockSpec((2, BN, HID), lambda i: (0, i, 0)),
            pl.BlockSpec((BN, HID), lambda i: (i, 0)),
            pl.BlockSpec((BN, 1), lambda i: (i, 0)),
            _full((1, HID)),
            pl.BlockSpec((BN, HID), lambda i: (i, 0)),
            _full((HID, HID)), _full((HID, HID)),
        ],
        out_specs=[pl.BlockSpec((BN, HID), lambda i: (i, 0))] * 2,
        out_shape=[jax.ShapeDtypeStruct((NP, HID), jnp.float32)] * 2,
    )(p, g, dv, b, xid, wa, wb)


def _head_body(uv_ref, b1_ref, w2_ref, b2_ref, w3_ref, b3_ref, o_ref):
    t = jnp.maximum(uv_ref[...] + b1_ref[...], 0.0)
    t2 = jnp.maximum(_dot(t, w2_ref[...]) + b2_ref[...] + t, 0.0)
    o_ref[...] = _dot(t2, w3_ref[...]) + b3_ref[...]


def _tc_head(uv, b1, w2, b2, w3, b3):
    return pl.pallas_call(
        _head_body,
        grid=(EP // BE,),
        in_specs=[
            pl.BlockSpec((BE, HID), lambda i: (i, 0)),
            _full((1, HID)), _full((HID, HID)), _full((1, HID)),
            _full((HID, 1)), _full((1, 1)),
        ],
        out_specs=pl.BlockSpec((BE, 1), lambda i: (i, 0)),
        out_shape=jax.ShapeDtypeStruct((EP, 1), jnp.float32),
    )(uv, b1, w2, b2, w3, b3)


def kernel(x, edge_attr, enn_w1, enn_b1, enn_w2, enn_b2, root_w, nn_bias,
           conv_ws, conv_bs, mlp_ws, mlp_bs, edge_index):
    f32 = jnp.float32
    src = edge_index[0]
    dst = edge_index[1]

    pad = EP - E
    src2 = jnp.concatenate([src, jnp.zeros((pad,), jnp.int32)]).reshape(EP // 128, 128)
    dst2 = jnp.concatenate([dst, jnp.full((pad,), NP - 1, jnp.int32)]).reshape(EP // 128, 128)
    ea_p = jnp.concatenate([edge_attr, jnp.zeros((pad, 3), f32)])

    x128 = jnp.zeros((NP, HID), f32).at[:N, :IN_CH].set(x)
    z128 = jnp.zeros((128, HID), f32)

    rw = jnp.zeros((HID, EMB), f32).at[:IN_CH].set(root_w)

    xs = _sc_gather128(x128, src2)
    msgs = _tc_msgs(ea_p, xs, enn_w1, enn_b1.reshape(1, 8), enn_w2,
                    enn_b2.reshape(1, IN_CH * EMB))
    pm = _sc_scatter128(msgs, dst2, z128).reshape(2, NP, HID)
    deg = pm[0, :, EMB] + pm[1, :, EMB] + 1.0
    dinv = (1.0 / jnp.sqrt(deg)).reshape(NP, 1)
    xc, g = _tc_node(pm, x128, rw, nn_bias.reshape(1, EMB), conv_ws[0], dinv)

    for i in range(NCONV):
        ps = _sc_gs(g, src2, dst2, z128).reshape(2, NP, HID)
        b = conv_bs[i].reshape(1, HID)
        xid = xc if i > 0 else None
        if i < NCONV - 1:
            xc, g = _tc_post_pre(ps, g, dinv, b, conv_ws[i + 1], xid)
        else:
            u, v = _tc_post_uv(ps, g, dinv, b, xid,
                               mlp_ws[0][:HID], mlp_ws[0][HID:])

    uv = _sc_gather_add128(u, v, src2, dst2)
    out = _tc_head(uv, mlp_bs[0].reshape(1, HID), mlp_ws[1],
                   mlp_bs[1].reshape(1, HID), mlp_ws[2], mlp_bs[2].reshape(1, 1))
    return out[:E]
